# baseline (speedup 1.0000x reference)
import functools

import jax
import jax.numpy as jnp
from jax import lax
from jax.experimental import pallas as pl
from jax.experimental.pallas import tpu as pltpu
from jax.experimental.pallas import tpu_sc as plsc

DIM = 768
N_EXPERTS = 64
TOP_K = 8
BLOCK_T = 2048
NWORKERS = 32
LANES = 16


def _scores_block(x_ref, w_ref, b_ref, s_ref):
    s_ref[...] = (jnp.dot(x_ref[...], w_ref[...],
                          preferred_element_type=jnp.float32) + b_ref[...])


def _tc_scores(xf, wt, bias):
    t = xf.shape[0]
    bt = min(BLOCK_T, t)
    return pl.pallas_call(
        _scores_block,
        grid=(pl.cdiv(t, bt),),
        in_specs=[
            pl.BlockSpec((bt, DIM), lambda i: (i, 0)),
            pl.BlockSpec((DIM, N_EXPERTS), lambda i: (0, 0)),
            pl.BlockSpec((1, N_EXPERTS), lambda i: (0, 0)),
        ],
        out_specs=pl.BlockSpec((bt, N_EXPERTS), lambda i: (i, 0)),
        out_shape=jax.ShapeDtypeStruct((t, N_EXPERTS), jnp.float32),
    )(xf, wt, bias)


def _merge16(ak, av, bk, bv):
    rbk = lax.rev(bk, (0,))
    rbv = lax.rev(bv, (0,))
    take = ak >= rbk
    mk = jnp.where(take, ak, rbk)
    mv = jnp.where(take, av, rbv)
    return plsc.sort_key_val(mk, mv, descending=True)


def _sc_router_body(s_hbm, w_hbm, i_hbm, sv, wv, iv, kscr, vscr):
    wid = lax.axis_index("s") * 2 + lax.axis_index("c")
    tpw = sv.shape[0] // N_EXPERTS
    base = wid * tpw
    pltpu.sync_copy(s_hbm.at[pl.ds(base * N_EXPERTS, tpw * N_EXPERTS)], sv)

    lane = lax.iota(jnp.int32, LANES)
    lo8 = lane < 8
    ninf = jnp.float32(-jnp.inf)

    def body(t, carry):
        srt = []
        for j in range(4):
            k = sv[pl.ds(t * N_EXPERTS + j * LANES, LANES)]
            srt.append(plsc.sort_key_val(k, lane + j * LANES, descending=True))
        m01k, m01v = _merge16(*srt[0], *srt[1])
        m23k, m23v = _merge16(*srt[2], *srt[3])
        kscr[pl.ds(0, LANES)] = lax.rev(m23k, (0,))
        vscr[pl.ds(0, LANES)] = lax.rev(m23v, (0,))
        xk = kscr[pl.ds(8, LANES)]
        xv = vscr[pl.ds(8, LANES)]
        take = m01k >= xk
        fk = jnp.where(take & lo8, m01k, jnp.where(lo8, xk, ninf))
        fv = jnp.where(take, m01v, xv)
        fk, fv = plsc.sort_key_val(fk, fv, descending=True)
        e = jnp.exp(fk - jnp.max(fk))
        w = e / (jnp.sum(e) + 1e-8)
        wv[pl.ds(t * LANES, LANES)] = w
        iv[pl.ds(t * LANES, LANES)] = fv
        return carry

    lax.fori_loop(0, tpw, body, 0)
    pltpu.sync_copy(wv, w_hbm.at[pl.ds(base * LANES, tpw * LANES)])
    pltpu.sync_copy(iv, i_hbm.at[pl.ds(base * LANES, tpw * LANES)])


def _sc_router(scores_flat, t):
    tpw = t // NWORKERS
    mesh = plsc.VectorSubcoreMesh(core_axis_name="c", subcore_axis_name="s")
    return pl.kernel(
        _sc_router_body,
        out_type=[
            jax.ShapeDtypeStruct((t * LANES,), jnp.float32),
            jax.ShapeDtypeStruct((t * LANES,), jnp.int32),
        ],
        mesh=mesh,
        scratch_types=[
            pltpu.VMEM((tpw * N_EXPERTS,), jnp.float32),
            pltpu.VMEM((tpw * LANES,), jnp.float32),
            pltpu.VMEM((tpw * LANES,), jnp.int32),
            pltpu.VMEM((32,), jnp.float32),
            pltpu.VMEM((32,), jnp.int32),
        ],
        compiler_params=pltpu.CompilerParams(needs_layout_passes=False),
    )(scores_flat)


@functools.partial(jax.jit, static_argnames=())
def kernel(x, gate_weight, adaptive_bias):
    orig_shape = x.shape
    xf = x.reshape(-1, orig_shape[-1])
    t = xf.shape[0]
    wt = gate_weight.T
    bias = adaptive_bias.reshape(1, N_EXPERTS)
    scores = _tc_scores(xf, wt, bias)
    w16, i16 = _sc_router(scores.reshape(-1), t)
    wts = w16.reshape(t, LANES)[:, :TOP_K]
    idx = i16.reshape(t, LANES)[:, :TOP_K]
    if len(orig_shape) == 3:
        wts = wts.reshape(orig_shape[0], orig_shape[1], TOP_K)
        idx = idx.reshape(orig_shape[0], orig_shape[1], TOP_K)
    return (wts, idx)


# speedup vs baseline: 1.7558x; 1.7558x over previous
import functools

import jax
import jax.numpy as jnp
from jax import lax
from jax.experimental import pallas as pl

DIM = 768
N_EXPERTS = 64
TOP_K = 8
BLOCK_T = 2048

_IDX_MASK = N_EXPERTS - 1


def _gate_block(x_ref, w_ref, b_ref, wout_ref, iout_ref):
    x = x_ref[...]
    w = w_ref[...]
    scores = jnp.dot(x, w, preferred_element_type=jnp.float32) + b_ref[...]
    niota = -lax.broadcasted_iota(jnp.int32, scores.shape, 1).astype(jnp.float32)
    oiota = lax.broadcasted_iota(jnp.int32, (scores.shape[0], TOP_K), 1)
    svals = jnp.zeros((scores.shape[0], TOP_K), jnp.float32)
    sneg = jnp.zeros((scores.shape[0], TOP_K), jnp.float32)
    m0 = None
    for k in range(TOP_K):
        m = jnp.max(scores, axis=-1, keepdims=True)
        cand = jnp.where(scores == m, niota, -jnp.inf)
        a = jnp.max(cand, axis=-1, keepdims=True)
        scores = jnp.where(cand == a, -jnp.inf, scores)
        svals = jnp.where(oiota == k, m, svals)
        sneg = jnp.where(oiota == k, a, sneg)
        if k == 0:
            m0 = m
    iout_ref[...] = (-sneg).astype(jnp.int32)
    e = jnp.exp(svals - m0)
    wout_ref[...] = e / (jnp.sum(e, axis=-1, keepdims=True) + 1e-8)


@functools.partial(jax.jit, static_argnames=())
def kernel(x, gate_weight, adaptive_bias):
    orig_shape = x.shape
    xf = x.reshape(-1, orig_shape[-1])
    t = xf.shape[0]
    bt = min(BLOCK_T, t)
    wt = gate_weight.T
    bias = adaptive_bias.reshape(1, N_EXPERTS)
    grid = (pl.cdiv(t, bt),)
    wts, idx = pl.pallas_call(
        _gate_block,
        grid=grid,
        in_specs=[
            pl.BlockSpec((bt, DIM), lambda i: (i, 0)),
            pl.BlockSpec((DIM, N_EXPERTS), lambda i: (0, 0)),
            pl.BlockSpec((1, N_EXPERTS), lambda i: (0, 0)),
        ],
        out_specs=[
            pl.BlockSpec((bt, TOP_K), lambda i: (i, 0)),
            pl.BlockSpec((bt, TOP_K), lambda i: (i, 0)),
        ],
        out_shape=[
            jax.ShapeDtypeStruct((t, TOP_K), jnp.float32),
            jax.ShapeDtypeStruct((t, TOP_K), jnp.int32),
        ],
    )(xf, wt, bias)
    if len(orig_shape) == 3:
        wts = wts.reshape(orig_shape[0], orig_shape[1], TOP_K)
        idx = idx.reshape(orig_shape[0], orig_shape[1], TOP_K)
    return (wts, idx)
